# baseline (speedup 1.0000x reference)
import jax
import jax.numpy as jnp
from jax.experimental import pallas as pl
from jax.experimental.pallas import tpu as pltpu

_TEMPERATURE = 0.1
_NUM_NEGATIVES = 16
_TN = 256


def _ctl_kernel(s_ref, tc_ref, cb_ref, out_ref):
    s = s_ref[...]
    cb = cb_ref[...]
    tc = tc_ref[...]
    tn, _ = s.shape
    k_dim = cb.shape[0]

    cb_sq = jnp.sum(cb * cb, axis=1, keepdims=True).T
    s_sq = jnp.sum(s * s, axis=1, keepdims=True)
    cross = jax.lax.dot_general(
        s, cb, (((1,), (1,)), ((), ())),
        preferred_element_type=jnp.float32)

    sq = s_sq + cb_sq - 2.0 * cross
    dist = jnp.maximum(sq, 0.0)
    col = jax.lax.broadcasted_iota(jnp.int32, (tn, k_dim), 1)
    posmask = col == tc
    inf = jnp.float32(jnp.inf)
    dist = jnp.where(posmask, inf, dist)

    work = dist
    m = jnp.min(work, axis=1, keepdims=True)
    for _ in range(_NUM_NEGATIVES - 1):
        work = jnp.where(work == m, inf, work)
        m = jnp.min(work, axis=1, keepdims=True)
    theta = m

    selmask = dist <= theta

    inv_cbn = 1.0 / jnp.maximum(jnp.sqrt(cb_sq), 1e-12)
    inv_snt = (1.0 / _TEMPERATURE) / jnp.maximum(
        jnp.sqrt(s_sq), 1e-12)
    simt = cross * inv_cbn * inv_snt

    z_neg = jnp.sum(jnp.where(selmask, jnp.exp(simt), 0.0),
                    axis=1, keepdims=True)
    pos_simt = jnp.sum(jnp.where(posmask, simt, 0.0),
                       axis=1, keepdims=True)
    ce = jnp.log(jnp.exp(pos_simt) + z_neg) - pos_simt
    out_ref[...] = jnp.sum(ce).reshape(1, 1, 1)


def kernel(student_features, teacher_codes, codebook):
    b, t, d_dim = student_features.shape
    n = b * t
    k_dim = codebook.shape[0]
    s_flat = student_features.reshape(n, d_dim)
    tc_flat = teacher_codes.reshape(n, 1).astype(jnp.int32)
    num_tiles = n // _TN

    partial = pl.pallas_call(
        _ctl_kernel,
        grid=(num_tiles,),
        in_specs=[
            pl.BlockSpec((_TN, d_dim), lambda i: (i, 0)),
            pl.BlockSpec((_TN, 1), lambda i: (i, 0)),
            pl.BlockSpec((k_dim, d_dim), lambda i: (0, 0)),
        ],
        out_specs=pl.BlockSpec((1, 1, 1), lambda i: (i, 0, 0)),
        out_shape=jax.ShapeDtypeStruct((num_tiles, 1, 1), jnp.float32),
        compiler_params=pltpu.CompilerParams(
            dimension_semantics=("parallel",)),
    )(s_flat, tc_flat, codebook)
    return jnp.sum(partial) / n


# speedup vs baseline: 1.2020x; 1.2020x over previous
import jax
import jax.numpy as jnp
from jax.experimental import pallas as pl
from jax.experimental.pallas import tpu as pltpu

_TEMPERATURE = 0.1
_NUM_NEGATIVES = 16
_TN = 256
_NB = 8

_NET8 = [(0, 1), (2, 3), (4, 5), (6, 7),
         (0, 2), (1, 3), (4, 6), (5, 7),
         (1, 2), (5, 6),
         (0, 4), (1, 5), (2, 6), (3, 7),
         (2, 4), (3, 5),
         (1, 2), (3, 4), (5, 6)]

_LOG2E = 1.4426950408889634
_LN2 = 0.6931471805599453


def _ctl_kernel(s_ref, tc_ref, cb_ref, out_ref):
    s = s_ref[...]
    cb = cb_ref[...]
    tc = tc_ref[...]
    tn, _ = s.shape
    k_dim = cb.shape[0]

    cb_sq = jnp.sum(cb * cb, axis=1, keepdims=True)
    s_sq = jnp.sum(s * s, axis=1, keepdims=True)
    ones_s = jnp.ones_like(s_sq)
    ones_c = jnp.ones_like(cb_sq)

    a_aug = jnp.concatenate([-2.0 * s, s_sq, ones_s], axis=1)
    b_aug = jnp.concatenate([cb, ones_c, cb_sq], axis=1)
    sq = jax.lax.dot_general(
        a_aug, b_aug, (((1,), (1,)), ((), ())),
        preferred_element_type=jnp.float32)

    dist = jnp.maximum(sq, 0.0)
    col = jax.lax.broadcasted_iota(jnp.int32, (tn, k_dim), 1)
    posmask = col == tc
    inf = jnp.float32(jnp.inf)
    dist = jnp.where(posmask, inf, dist)

    w = k_dim // _NB
    blk = [dist[:, j * w:(j + 1) * w] for j in range(_NB)]
    for a, b in _NET8:
        lo = jnp.minimum(blk[a], blk[b])
        hi = jnp.maximum(blk[a], blk[b])
        blk[a], blk[b] = lo, hi
    m = jnp.min(blk[0], axis=1, keepdims=True)
    for _ in range(_NUM_NEGATIVES - 1):
        mask = blk[0] == m
        for j in range(_NB - 1):
            blk[j] = jnp.where(mask, blk[j + 1], blk[j])
        blk[_NB - 1] = jnp.where(mask, inf, blk[_NB - 1])
        m = jnp.min(blk[0], axis=1, keepdims=True)
    theta = m

    selmask = dist <= theta

    inv_snt = (_LOG2E / _TEMPERATURE) / jnp.maximum(
        jnp.sqrt(s_sq), 1e-12)
    inv_cbn = 1.0 / jnp.maximum(jnp.sqrt(cb_sq), 1e-12)
    y = jax.lax.dot_general(
        s * inv_snt, cb * inv_cbn, (((1,), (1,)), ((), ())),
        preferred_element_type=jnp.float32)

    z_neg = jnp.sum(jnp.where(selmask, jnp.exp2(y), 0.0),
                    axis=1, keepdims=True)
    p2 = jnp.sum(jnp.where(posmask, y, 0.0),
                 axis=1, keepdims=True)
    ce = _LN2 * (jnp.log2(jnp.exp2(p2) + z_neg) - p2)
    out_ref[...] = jnp.sum(ce).reshape(1, 1, 1)


def kernel(student_features, teacher_codes, codebook):
    b, t, d_dim = student_features.shape
    n = b * t
    k_dim = codebook.shape[0]
    s_flat = student_features.reshape(n, d_dim)
    tc_flat = teacher_codes.reshape(n, 1).astype(jnp.int32)
    num_tiles = n // _TN

    partial = pl.pallas_call(
        _ctl_kernel,
        grid=(num_tiles,),
        in_specs=[
            pl.BlockSpec((_TN, d_dim), lambda i: (i, 0)),
            pl.BlockSpec((_TN, 1), lambda i: (i, 0)),
            pl.BlockSpec((k_dim, d_dim), lambda i: (0, 0)),
        ],
        out_specs=pl.BlockSpec((1, 1, 1), lambda i: (i, 0, 0)),
        out_shape=jax.ShapeDtypeStruct((num_tiles, 1, 1), jnp.float32),
        compiler_params=pltpu.CompilerParams(
            dimension_semantics=("parallel",)),
    )(s_flat, tc_flat, codebook)
    return jnp.sum(partial) / n


# speedup vs baseline: 1.3322x; 1.1083x over previous
import jax
import jax.numpy as jnp
from jax.experimental import pallas as pl
from jax.experimental.pallas import tpu as pltpu

_TEMPERATURE = 0.1
_NUM_NEGATIVES = 16
_TN = 256
_NB = 8

_NET8 = [(0, 1), (2, 3), (4, 5), (6, 7),
         (0, 2), (1, 3), (4, 6), (5, 7),
         (1, 2), (5, 6),
         (0, 4), (1, 5), (2, 6), (3, 7),
         (2, 4), (3, 5),
         (1, 2), (3, 4), (5, 6)]

_LOG2E = 1.4426950408889634
_LN2 = 0.6931471805599453


def _ctl_kernel(s_ref, tc_ref, cb_ref, out_ref, baug_ref, cbn_ref):
    i = pl.program_id(0)
    s = s_ref[...]
    tc = tc_ref[...]
    tn, d_dim = s.shape
    k_dim = baug_ref.shape[0]

    @pl.when(i == 0)
    def _prep():
        cb = cb_ref[...]
        cb_sq = jnp.sum(cb * cb, axis=1, keepdims=True)
        baug_ref[...] = jnp.concatenate(
            [cb, jnp.ones_like(cb_sq), cb_sq], axis=1)
        inv_cbn = 1.0 / jnp.maximum(jnp.sqrt(cb_sq), 1e-12)
        cbn_ref[...] = cb * inv_cbn

    s_sq = jnp.sum(s * s, axis=1, keepdims=True)

    a_aug = jnp.concatenate(
        [-2.0 * s, s_sq, jnp.ones_like(s_sq)], axis=1)
    sq = jax.lax.dot_general(
        a_aug, baug_ref[...], (((1,), (1,)), ((), ())),
        preferred_element_type=jnp.float32)

    col = jax.lax.broadcasted_iota(jnp.int32, (tn, k_dim), 1)
    posmask = col == tc
    inf = jnp.float32(jnp.inf)
    dist = jnp.where(posmask, inf, jnp.maximum(sq, 0.0))

    w = k_dim // _NB
    blk = [dist[:, j * w:(j + 1) * w] for j in range(_NB)]
    for a, b in _NET8:
        lo = jnp.minimum(blk[a], blk[b])
        hi = jnp.maximum(blk[a], blk[b])
        blk[a], blk[b] = lo, hi
    m = jnp.min(blk[0], axis=1, keepdims=True)
    for _ in range(_NUM_NEGATIVES - 1):
        mask = blk[0] == m
        for j in range(_NB - 1):
            blk[j] = jnp.where(mask, blk[j + 1], blk[j])
        blk[_NB - 1] = jnp.where(mask, inf, blk[_NB - 1])
        m = jnp.min(blk[0], axis=1, keepdims=True)
    theta = m

    selmask = dist <= theta

    inv_snt = (_LOG2E / _TEMPERATURE) / jnp.maximum(
        jnp.sqrt(s_sq), 1e-12)
    y = jax.lax.dot_general(
        s * inv_snt, cbn_ref[...], (((1,), (1,)), ((), ())),
        preferred_element_type=jnp.float32)

    z_neg = jnp.sum(jnp.where(selmask, jnp.exp2(y), 0.0),
                    axis=1, keepdims=True)
    p2 = jnp.sum(jnp.where(posmask, y, 0.0),
                 axis=1, keepdims=True)
    ce = _LN2 * (jnp.log2(jnp.exp2(p2) + z_neg) - p2)
    out_ref[...] = jnp.sum(ce).reshape(1, 1, 1)


def kernel(student_features, teacher_codes, codebook):
    b, t, d_dim = student_features.shape
    n = b * t
    k_dim = codebook.shape[0]
    s_flat = student_features.reshape(n, d_dim)
    tc_flat = teacher_codes.reshape(n, 1).astype(jnp.int32)
    num_tiles = n // _TN

    partial = pl.pallas_call(
        _ctl_kernel,
        grid=(num_tiles,),
        in_specs=[
            pl.BlockSpec((_TN, d_dim), lambda i: (i, 0)),
            pl.BlockSpec((_TN, 1), lambda i: (i, 0)),
            pl.BlockSpec((k_dim, d_dim), lambda i: (0, 0)),
        ],
        out_specs=pl.BlockSpec((1, 1, 1), lambda i: (i, 0, 0)),
        out_shape=jax.ShapeDtypeStruct((num_tiles, 1, 1), jnp.float32),
        scratch_shapes=[
            pltpu.VMEM((k_dim, d_dim + 2), jnp.float32),
            pltpu.VMEM((k_dim, d_dim), jnp.float32),
        ],
        compiler_params=pltpu.CompilerParams(
            dimension_semantics=("arbitrary",)),
    )(s_flat, tc_flat, codebook)
    return jnp.sum(partial) / n
